# baseline (speedup 1.0000x reference)
import functools

import jax
import jax.numpy as jnp
from jax import lax
from jax.experimental import pallas as pl
from jax.experimental.pallas import tpu as pltpu
from jax.experimental.pallas import tpu_sc as plsc

N = 20000
D = 64
U = 2 * N
CHUNK = 128
NPAD = 20480
NCH = NPAD // CHUNK
NTILES = 16
CPT = NCH // NTILES
PROWS = 8192
SEGR = PROWS // NTILES
SREP = SEGR // CHUNK
TRASHR = 7816
TOTROW = 8000
HALFW = 20480
PK = HALFW // 2
ACC_T = PK
MROWS = 168
MTRASH = 160
THR = 0.5


def _rank_body(xc, mc, invx, invm,
               cbuf, gbuf, sbuf, obuf, scbuf, grows, obuf1, sidx, p2):
  c = lax.axis_index("c")
  s = lax.axis_index("s")
  iot = lax.iota(jnp.int32, 16)
  z16 = jnp.zeros((16,), jnp.int32)
  one16 = jnp.full((16,), 1, jnp.int32)

  def _ld(i, _):
    ch = s * CPT + i
    pltpu.sync_copy(xc.at[pl.ds(ch * CHUNK, CHUNK)], cbuf.at[i])
    pltpu.sync_copy(mc.at[pl.ds(ch * CHUNK, CHUNK)], cbuf.at[CPT + i])
    return 0
  lax.fori_loop(0, CPT, _ld, 0)

  def _rows(k, _):
    ch = k // 8
    off = (k % 8) * 16
    v = cbuf[ch, pl.ds(off, 16)]
    sbuf[ch, pl.ds(off, 16)] = jnp.where(v < 0, TRASHR + s, v >> 7)
    gbuf[ch, pl.ds(off, 16)] = jnp.maximum(v, 0) >> 7
    return 0
  lax.fori_loop(0, 2 * CPT * 8, _rows, 0)

  def _seg_idx(k):
    def _bi(j, _):
      sidx[pl.ds(j * 16, 16)] = s * SEGR + k * CHUNK + iot + j * 16
      return 0
    lax.fori_loop(0, 8, _bi, 0)

  def _zs(j, _):
    scbuf[j // 8, pl.ds((j % 8) * 16, 16)] = z16
    return 0
  lax.fori_loop(0, CHUNK * 8, _zs, 0, unroll=8)

  def _zp(k, _):
    _seg_idx(k)
    pltpu.sync_copy(scbuf, p2.at[sidx])
    return 0
  lax.fori_loop(0, SREP, _zp, 0)

  def _zo(j, _):
    obuf[j // 8, pl.ds((j % 8) * 16, 16)] = z16
    return 0
  lax.fori_loop(0, CHUNK * 8, _zo, 0, unroll=8)
  plsc.subcore_barrier()

  def _pres(ch, _):
    def _set(j, _2):
      col = cbuf[ch, pl.ds(j * 16, 16)] & 127
      plsc.store_scatter(obuf, [iot + j * 16, col], one16)
      return 0
    lax.fori_loop(0, CHUNK // 16, _set, 0, unroll=8)
    pltpu.sync_copy(obuf, p2.at[sbuf.at[ch]], add=True)

    def _unset(j, _2):
      col = cbuf[ch, pl.ds(j * 16, 16)] & 127
      plsc.store_scatter(obuf, [iot + j * 16, col], z16)
      return 0
    lax.fori_loop(0, CHUNK // 16, _unset, 0, unroll=8)
    return 0
  lax.fori_loop(0, 2 * CPT, _pres, 0)
  plsc.subcore_barrier()

  def _tot(k, t):
    _seg_idx(k)
    pltpu.sync_copy(p2.at[sidx], scbuf)

    def _acc(j, a):
      v = scbuf[j // 8, pl.ds((j % 8) * 16, 16)]
      return a + jnp.minimum(v, 1)
    accv = lax.fori_loop(0, CHUNK * 8, _acc, z16, unroll=8)
    return t + jnp.sum(accv)
  total = lax.fori_loop(0, SREP, _tot, jnp.int32(0))

  plsc.store_scatter(obuf, [z16, jnp.full((16,), s, jnp.int32)],
                     jnp.full((16,), total, jnp.int32))

  def _ti(j, _):
    flat = iot + j * 16
    sidx[pl.ds(j * 16, 16)] = jnp.where(flat == 0, TOTROW, TRASHR)
    return 0
  lax.fori_loop(0, 8, _ti, 0)
  pltpu.sync_copy(obuf, p2.at[sidx], add=True)
  plsc.subcore_barrier()
  pltpu.sync_copy(p2.at[pl.ds(TOTROW, CHUNK)], scbuf)
  tot16 = scbuf[0, pl.ds(0, 16)]
  exc = plsc.cumsum(tot16) - tot16
  base = jnp.sum(jnp.where(iot == s, exc, 0))

  def _p2k(k, carry_in):
    _seg_idx(k)
    pltpu.sync_copy(p2.at[sidx], scbuf)

    def _scan(j, carry):
      v = jnp.minimum(scbuf[j // 8, pl.ds((j % 8) * 16, 16)], 1)
      inc = plsc.cumsum(v)
      scbuf[j // 8, pl.ds((j % 8) * 16, 16)] = inc - v + carry
      return carry + inc[15]
    carry_out = lax.fori_loop(0, CHUNK * 8, _scan, carry_in, unroll=4)
    pltpu.sync_copy(scbuf, p2.at[sidx])
    return carry_out
  lax.fori_loop(0, SREP, _p2k, base)
  plsc.subcore_barrier()

  def _emit(ch, gch, out_ref):
    pltpu.sync_copy(p2.at[gbuf.at[ch]], grows)

    def _ext(j, _):
      col = jnp.maximum(cbuf[ch, pl.ds(j * 16, 16)], 0) & 127
      obuf1[pl.ds(j * 16, 16)] = plsc.load_gather(grows, [iot + j * 16, col])
      return 0
    lax.fori_loop(0, CHUNK // 16, _ext, 0)
    pltpu.sync_copy(obuf1, out_ref.at[pl.ds((s * CPT + gch) * CHUNK, CHUNK)])

  @pl.when(c == 0)
  def _():
    def _ex(i, _):
      _emit(i, i, invx)
      return 0
    lax.fori_loop(0, CPT, _ex, 0)

  @pl.when(c == 1)
  def _():
    def _em(i, _):
      _emit(CPT + i, i, invm)
      return 0
    lax.fori_loop(0, CPT, _em, 0)


@functools.cache
def _rank_kernel():
  mesh = plsc.VectorSubcoreMesh(core_axis_name="c", subcore_axis_name="s")
  return pl.kernel(
      _rank_body,
      out_type=[
          jax.ShapeDtypeStruct((NPAD,), jnp.int32),
          jax.ShapeDtypeStruct((NPAD,), jnp.int32),
      ],
      mesh=mesh,
      compiler_params=pltpu.CompilerParams(needs_layout_passes=False),
      scratch_types=[
          pltpu.VMEM((2 * CPT, CHUNK), jnp.int32),
          pltpu.VMEM((2 * CPT, CHUNK), jnp.int32),
          pltpu.VMEM((2 * CPT, CHUNK), jnp.int32),
          pltpu.VMEM((CHUNK, CHUNK), jnp.int32),
          pltpu.VMEM((CHUNK, CHUNK), jnp.int32),
          pltpu.VMEM((CHUNK, CHUNK), jnp.int32),
          pltpu.VMEM((CHUNK,), jnp.int32),
          pltpu.VMEM((CHUNK,), jnp.int32),
          pltpu.VMEM_SHARED((PROWS, CHUNK), jnp.int32),
      ],
      name="sc_coord_rank",
  )


def _scatter_body(xf, mf, invx, invm, xexp, msc,
                  ibuf, xrows, wl, wr, mvals, sidx, sidx2, acc, macc):
  c = lax.axis_index("c")
  s = lax.axis_index("s")
  iot = lax.iota(jnp.int32, 16)
  zf16 = jnp.zeros((16,), jnp.float32)

  def _ld(i, _):
    ch = s * CPT + i
    pltpu.sync_copy(invx.at[pl.ds(ch * CHUNK, CHUNK)], ibuf.at[i])
    pltpu.sync_copy(invm.at[pl.ds(ch * CHUNK, CHUNK)], ibuf.at[CPT + i])
    return 0
  lax.fori_loop(0, CPT, _ld, 0)
  base_row = c * HALFW

  def _locm(k, _):
    i = k // 8
    off = (k % 8) * 16
    r = ibuf[CPT + i, pl.ds(off, 16)] - base_row
    ok = (r >= 0) & (r < HALFW)
    ibuf[CPT + i, pl.ds(off, 16)] = jnp.where(ok, r, -2)
    return 0
  lax.fori_loop(0, CPT * 8, _locm, 0)

  def _zw(j, _):
    wl[j // 8, pl.ds((j % 8) * 16, 16)] = zf16
    wr[j // 8, pl.ds((j % 8) * 16, 16)] = zf16
    return 0
  lax.fori_loop(0, CHUNK * 8, _zw, 0, unroll=8)

  def _za(k, _):
    def _bi(j, _2):
      sidx[pl.ds(j * 16, 16)] = s * 640 + k * CHUNK + iot + j * 16
      return 0
    lax.fori_loop(0, 8, _bi, 0)
    pltpu.sync_copy(wl, acc.at[sidx])
    return 0
  lax.fori_loop(0, PK // (NTILES * CHUNK), _za, 0)

  @pl.when(s == 0)
  def _():
    def _bi(j, _2):
      sidx[pl.ds(j * 16, 16)] = jnp.minimum(PK + iot + j * 16, PK + 7)
      return 0
    lax.fori_loop(0, 8, _bi, 0)
    pltpu.sync_copy(wl, acc.at[sidx])

  @pl.when(s == 1)
  def _():
    pltpu.sync_copy(wl, macc.at[pl.ds(0, CHUNK)])
    pltpu.sync_copy(wl, macc.at[pl.ds(40, CHUNK)])
  plsc.subcore_barrier()

  def _sx(t, _):
    i = t // 2
    u = t % 2
    ch = s * CPT + i
    pltpu.sync_copy(xf.at[pl.ds(ch * CHUNK + u * 64, 64)], xrows)

    def _cp(k, _2):
      j = k // 4
      q = (k % 4) * 16
      v = xrows[j, pl.ds(q, 16)]
      wl[j, pl.ds(q, 16)] = v
      wr[j, pl.ds(D + q, 16)] = v
      return 0
    lax.fori_loop(0, 64 * 4, _cp, 0, unroll=8)

    def _bi(j, _2):
      r = ibuf[i, pl.ds(u * 64 + j * 16, 16)] - base_row
      ok = (r >= 0) & (r < HALFW)
      sidx2[pl.ds(j * 16, 16)] = jnp.where(
          ok & ((r & 1) == 0), r >> 1, ACC_T)
      return 0
    lax.fori_loop(0, 4, _bi, 0)
    pltpu.sync_copy(wl.at[pl.ds(0, 64)], acc.at[sidx2], add=True)

    def _bo(j, _2):
      r = ibuf[i, pl.ds(u * 64 + j * 16, 16)] - base_row
      ok = (r >= 0) & (r < HALFW)
      sidx2[pl.ds(j * 16, 16)] = jnp.where(
          ok & ((r & 1) == 1), r >> 1, ACC_T)
      return 0
    lax.fori_loop(0, 4, _bo, 0)
    pltpu.sync_copy(wr.at[pl.ds(0, 64)], acc.at[sidx2], add=True)
    return 0
  lax.fori_loop(0, 2 * CPT, _sx, 0)

  def _zw2(j, _):
    wr[j // 8, pl.ds((j % 8) * 16, 16)] = zf16
    return 0
  lax.fori_loop(0, CHUNK * 8, _zw2, 0, unroll=8)

  def _sm(i, _):
    ch = s * CPT + i
    pltpu.sync_copy(mf.at[pl.ds(ch * CHUNK, CHUNK)], mvals)

    def _bi(j, _2):
      lr = ibuf[CPT + i, pl.ds(j * 16, 16)]
      sidx[pl.ds(j * 16, 16)] = jnp.where(
          lr < 0, MTRASH, 80 * (lr & 1) + (lr >> 8))
      return 0
    lax.fori_loop(0, 8, _bi, 0)

    def _st(j, _2):
      lr = ibuf[CPT + i, pl.ds(j * 16, 16)]
      col = (jnp.maximum(lr, 0) >> 1) & 127
      plsc.store_scatter(wr, [iot + j * 16, col], mvals[pl.ds(j * 16, 16)])
      return 0
    lax.fori_loop(0, CHUNK // 16, _st, 0, unroll=8)
    pltpu.sync_copy(wr, macc.at[sidx], add=True)

    def _un(j, _2):
      lr = ibuf[CPT + i, pl.ds(j * 16, 16)]
      col = (jnp.maximum(lr, 0) >> 1) & 127
      plsc.store_scatter(wr, [iot + j * 16, col], zf16)
      return 0
    lax.fori_loop(0, CHUNK // 16, _un, 0, unroll=8)
    return 0
  lax.fori_loop(0, CPT, _sm, 0)
  plsc.subcore_barrier()

  def _out(k, _):
    def _bi(j, _2):
      sidx[pl.ds(j * 16, 16)] = s * 640 + k * CHUNK + iot + j * 16
      return 0
    lax.fori_loop(0, 8, _bi, 0)
    pltpu.sync_copy(acc.at[sidx], wl)
    pltpu.sync_copy(
        wl, xexp.at[pl.ds(c * PK + s * 640 + k * CHUNK, CHUNK)])
    return 0
  lax.fori_loop(0, PK // (NTILES * CHUNK), _out, 0)

  @pl.when(s == 0)
  def _():
    pltpu.sync_copy(macc.at[pl.ds(0, CHUNK)], wr)
    pltpu.sync_copy(wr, msc.at[pl.ds(c * 160, CHUNK)])
    pltpu.sync_copy(macc.at[pl.ds(32, CHUNK)], wr)
    pltpu.sync_copy(wr, msc.at[pl.ds(c * 160 + 32, CHUNK)])


@functools.cache
def _scatter_kernel():
  mesh = plsc.VectorSubcoreMesh(core_axis_name="c", subcore_axis_name="s")
  return pl.kernel(
      _scatter_body,
      out_type=[
          jax.ShapeDtypeStruct((2 * PK, CHUNK), jnp.float32),
          jax.ShapeDtypeStruct((320, CHUNK), jnp.float32),
      ],
      mesh=mesh,
      compiler_params=pltpu.CompilerParams(needs_layout_passes=False),
      scratch_types=[
          pltpu.VMEM((2 * CPT, CHUNK), jnp.int32),
          pltpu.VMEM((64, D), jnp.float32),
          pltpu.VMEM((CHUNK, CHUNK), jnp.float32),
          pltpu.VMEM((CHUNK, CHUNK), jnp.float32),
          pltpu.VMEM((CHUNK,), jnp.float32),
          pltpu.VMEM((CHUNK,), jnp.int32),
          pltpu.VMEM((64,), jnp.int32),
          pltpu.VMEM_SHARED((PK + 8, CHUNK), jnp.float32),
          pltpu.VMEM_SHARED((MROWS, CHUNK), jnp.float32),
      ],
      name="sc_union_scatter",
  )


def _fin_body(xe_ref, ms_ref, xp_ref, tg_ref):
  x = xe_ref[...]
  xe = x[:, 0:D]
  xo = x[:, D:2 * D]
  se = ms_ref[:, 0:1]
  so = ms_ref[:, 1:2]
  te = jnp.where((se > THR) & (jnp.max(xe, axis=1, keepdims=True) > 0.0),
                 1.0, 0.0)
  to = jnp.where((so > THR) & (jnp.max(xo, axis=1, keepdims=True) > 0.0),
                 1.0, 0.0)
  xp_ref[:, 0:D] = xe * te
  xp_ref[:, D:2 * D] = xo * to
  tg_ref[:, 0:1] = te
  tg_ref[:, 1:2] = to


def _finalize(xexp_p, msc_r):
  blk = 1280
  return pl.pallas_call(
      _fin_body,
      grid=(2 * PK // blk,),
      in_specs=[
          pl.BlockSpec((blk, CHUNK), lambda i: (i, 0)),
          pl.BlockSpec((blk, 2), lambda i: (i, 0)),
      ],
      out_specs=[
          pl.BlockSpec((blk, CHUNK), lambda i: (i, 0)),
          pl.BlockSpec((blk, 2), lambda i: (i, 0)),
      ],
      out_shape=[
          jax.ShapeDtypeStruct((2 * PK, CHUNK), jnp.float32),
          jax.ShapeDtypeStruct((2 * PK, 2), jnp.float32),
      ],
      compiler_params=pltpu.CompilerParams(
          dimension_semantics=("arbitrary",)),
  )(xexp_p, msc_r)


def kernel(x_feats, x_coords, mask_feats, mask_coords):
  pad = NPAD - N
  xf = jnp.pad(x_feats, ((0, pad), (0, 0)))
  xc = jnp.pad(x_coords, (0, pad), constant_values=-1)
  mf = jnp.pad(mask_feats.reshape(-1), (0, pad))
  mc = jnp.pad(mask_coords, (0, pad), constant_values=-1)
  invx, invm = _rank_kernel()(xc, mc)
  xexp_p, msc = _scatter_kernel()(xf, mf, invx, invm)
  msc_r = msc.reshape(2, 2, PK).transpose(0, 2, 1).reshape(2 * PK, 2)
  xp_pk, tg2 = _finalize(xexp_p, msc_r)
  xp = xp_pk.reshape(4 * PK, D)[:U]
  tg = tg2.reshape(4 * PK)[:U].astype(jnp.bool_)
  return xp, tg


# speedup vs baseline: 1.0096x; 1.0096x over previous
import functools

import jax
import jax.numpy as jnp
from jax import lax
from jax.experimental import pallas as pl
from jax.experimental.pallas import tpu as pltpu
from jax.experimental.pallas import tpu_sc as plsc

N = 20000
D = 64
U = 2 * N
CHUNK = 128
NPAD = 20480
NCH = NPAD // CHUNK
NTILES = 16
CPT = NCH // NTILES
PROWS = 8192
SEGR = PROWS // NTILES
SREP = SEGR // CHUNK
TRASHR = 7816
TOTROW = 8000
HALFW = 20480
PK = HALFW // 2
ACC_T = PK
MROWS = 168
MTRASH = 160
THR = 0.5


def _rank_body(xc, mc, invx, invm,
               cbuf, gbuf, sbuf, obuf, scbuf, grows, obuf1, sidx, p2):
  c = lax.axis_index("c")
  s = lax.axis_index("s")
  iot = lax.iota(jnp.int32, 16)
  z16 = jnp.zeros((16,), jnp.int32)
  one16 = jnp.full((16,), 1, jnp.int32)

  def _ld(i, _):
    ch = s * CPT + i
    pltpu.sync_copy(xc.at[pl.ds(ch * CHUNK, CHUNK)], cbuf.at[i])
    pltpu.sync_copy(mc.at[pl.ds(ch * CHUNK, CHUNK)], cbuf.at[CPT + i])
    return 0
  lax.fori_loop(0, CPT, _ld, 0)

  def _rows(k, _):
    ch = k // 8
    off = (k % 8) * 16
    v = cbuf[ch, pl.ds(off, 16)]
    sbuf[ch, pl.ds(off, 16)] = jnp.where(v < 0, TRASHR + s, v >> 7)
    gbuf[ch, pl.ds(off, 16)] = jnp.maximum(v, 0) >> 7
    return 0
  lax.fori_loop(0, 2 * CPT * 8, _rows, 0)

  def _seg_idx(k):
    def _bi(j, _):
      sidx[pl.ds(j * 16, 16)] = s * SEGR + k * CHUNK + iot + j * 16
      return 0
    lax.fori_loop(0, 8, _bi, 0)

  def _zs(j, _):
    scbuf[j // 8, pl.ds((j % 8) * 16, 16)] = z16
    return 0
  lax.fori_loop(0, CHUNK * 8, _zs, 0, unroll=8)

  def _zp(k, _):
    _seg_idx(k)
    pltpu.sync_copy(scbuf, p2.at[sidx])
    return 0
  lax.fori_loop(0, SREP, _zp, 0)

  def _zo(j, _):
    obuf[j // 8, pl.ds((j % 8) * 16, 16)] = z16
    return 0
  lax.fori_loop(0, CHUNK * 8, _zo, 0, unroll=8)
  plsc.subcore_barrier()

  def _pres(ch, _):
    def _set(j, _2):
      col = cbuf[ch, pl.ds(j * 16, 16)] & 127
      plsc.store_scatter(obuf, [iot + j * 16, col], one16)
      return 0
    lax.fori_loop(0, CHUNK // 16, _set, 0, unroll=8)
    pltpu.sync_copy(obuf, p2.at[sbuf.at[ch]], add=True)

    def _unset(j, _2):
      col = cbuf[ch, pl.ds(j * 16, 16)] & 127
      plsc.store_scatter(obuf, [iot + j * 16, col], z16)
      return 0
    lax.fori_loop(0, CHUNK // 16, _unset, 0, unroll=8)
    return 0
  lax.fori_loop(0, 2 * CPT, _pres, 0)
  plsc.subcore_barrier()

  def _tot(k, t):
    _seg_idx(k)
    pltpu.sync_copy(p2.at[sidx], scbuf)

    def _acc(j, a):
      v = scbuf[j // 8, pl.ds((j % 8) * 16, 16)]
      return a + jnp.minimum(v, 1)
    accv = lax.fori_loop(0, CHUNK * 8, _acc, z16, unroll=8)
    return t + jnp.sum(accv)
  total = lax.fori_loop(0, SREP, _tot, jnp.int32(0))

  plsc.store_scatter(obuf, [z16, jnp.full((16,), s, jnp.int32)],
                     jnp.full((16,), total, jnp.int32))

  def _ti(j, _):
    flat = iot + j * 16
    sidx[pl.ds(j * 16, 16)] = jnp.where(flat == 0, TOTROW, TRASHR)
    return 0
  lax.fori_loop(0, 8, _ti, 0)
  pltpu.sync_copy(obuf, p2.at[sidx], add=True)
  plsc.subcore_barrier()
  pltpu.sync_copy(p2.at[pl.ds(TOTROW, CHUNK)], scbuf)
  tot16 = scbuf[0, pl.ds(0, 16)]
  exc = plsc.cumsum(tot16) - tot16
  base = jnp.sum(jnp.where(iot == s, exc, 0))

  def _p2k(k, carry_in):
    _seg_idx(k)
    pltpu.sync_copy(p2.at[sidx], scbuf)

    def _scan(j, carry):
      v = jnp.minimum(scbuf[j // 8, pl.ds((j % 8) * 16, 16)], 1)
      inc = plsc.cumsum(v)
      scbuf[j // 8, pl.ds((j % 8) * 16, 16)] = inc - v + carry
      return carry + inc[15]
    carry_out = lax.fori_loop(0, CHUNK * 8, _scan, carry_in, unroll=4)
    pltpu.sync_copy(scbuf, p2.at[sidx])
    return carry_out
  lax.fori_loop(0, SREP, _p2k, base)
  plsc.subcore_barrier()

  def _emit(ch, gch, out_ref):
    pltpu.sync_copy(p2.at[gbuf.at[ch]], grows)

    def _ext(j, _):
      col = jnp.maximum(cbuf[ch, pl.ds(j * 16, 16)], 0) & 127
      obuf1[pl.ds(j * 16, 16)] = plsc.load_gather(grows, [iot + j * 16, col])
      return 0
    lax.fori_loop(0, CHUNK // 16, _ext, 0)
    pltpu.sync_copy(obuf1, out_ref.at[pl.ds((s * CPT + gch) * CHUNK, CHUNK)])

  @pl.when(c == 0)
  def _():
    def _ex(i, _):
      _emit(i, i, invx)
      return 0
    lax.fori_loop(0, CPT, _ex, 0)

  @pl.when(c == 1)
  def _():
    def _em(i, _):
      _emit(CPT + i, i, invm)
      return 0
    lax.fori_loop(0, CPT, _em, 0)


@functools.cache
def _rank_kernel():
  mesh = plsc.VectorSubcoreMesh(core_axis_name="c", subcore_axis_name="s")
  return pl.kernel(
      _rank_body,
      out_type=[
          jax.ShapeDtypeStruct((NPAD,), jnp.int32),
          jax.ShapeDtypeStruct((NPAD,), jnp.int32),
      ],
      mesh=mesh,
      compiler_params=pltpu.CompilerParams(needs_layout_passes=False),
      scratch_types=[
          pltpu.VMEM((2 * CPT, CHUNK), jnp.int32),
          pltpu.VMEM((2 * CPT, CHUNK), jnp.int32),
          pltpu.VMEM((2 * CPT, CHUNK), jnp.int32),
          pltpu.VMEM((CHUNK, CHUNK), jnp.int32),
          pltpu.VMEM((CHUNK, CHUNK), jnp.int32),
          pltpu.VMEM((CHUNK, CHUNK), jnp.int32),
          pltpu.VMEM((CHUNK,), jnp.int32),
          pltpu.VMEM((CHUNK,), jnp.int32),
          pltpu.VMEM_SHARED((PROWS, CHUNK), jnp.int32),
      ],
      name="sc_coord_rank",
  )


def _scatter_body(xf, mf, invx, invm, xexp, msc,
                  ibuf, xrows, wl, wr, mvals, sidx, sidx2, acc, macc):
  c = lax.axis_index("c")
  s = lax.axis_index("s")
  iot = lax.iota(jnp.int32, 16)
  zf16 = jnp.zeros((16,), jnp.float32)

  def _ld(i, _):
    ch = s * CPT + i
    pltpu.sync_copy(invx.at[pl.ds(ch * CHUNK, CHUNK)], ibuf.at[i])
    pltpu.sync_copy(invm.at[pl.ds(ch * CHUNK, CHUNK)], ibuf.at[CPT + i])
    return 0
  lax.fori_loop(0, CPT, _ld, 0)
  base_row = c * HALFW

  def _locm(k, _):
    i = k // 8
    off = (k % 8) * 16
    r = ibuf[CPT + i, pl.ds(off, 16)] - base_row
    ok = (r >= 0) & (r < HALFW)
    ibuf[CPT + i, pl.ds(off, 16)] = jnp.where(ok, r, -2)
    return 0
  lax.fori_loop(0, CPT * 8, _locm, 0)

  def _zw(j, _):
    wl[j // 8, pl.ds((j % 8) * 16, 16)] = zf16
    wr[j // 8, pl.ds((j % 8) * 16, 16)] = zf16
    return 0
  lax.fori_loop(0, CHUNK * 8, _zw, 0, unroll=8)

  def _za(k, _):
    def _bi(j, _2):
      sidx[pl.ds(j * 16, 16)] = s * 640 + k * CHUNK + iot + j * 16
      return 0
    lax.fori_loop(0, 8, _bi, 0)
    pltpu.sync_copy(wl, acc.at[sidx])
    return 0
  lax.fori_loop(0, PK // (NTILES * CHUNK), _za, 0)

  @pl.when(s == 0)
  def _():
    def _bi(j, _2):
      sidx[pl.ds(j * 16, 16)] = jnp.minimum(PK + iot + j * 16, PK + 7)
      return 0
    lax.fori_loop(0, 8, _bi, 0)
    pltpu.sync_copy(wl, acc.at[sidx])

  @pl.when(s == 1)
  def _():
    pltpu.sync_copy(wl, macc.at[pl.ds(0, CHUNK)])
    pltpu.sync_copy(wl, macc.at[pl.ds(40, CHUNK)])
  plsc.subcore_barrier()

  def _sx(t, _):
    i = t // 2
    u = t % 2
    ch = s * CPT + i
    start = ch * CHUNK + u * 64

    @pl.when(start <= N - 64)
    def _():
      pltpu.sync_copy(xf.at[pl.ds(start, 64)], xrows)

    @pl.when(start == N - 32)
    def _():
      pltpu.sync_copy(xf.at[pl.ds(N - 32, 32)], xrows.at[pl.ds(0, 32)])

      def _zt(k2, _3):
        xrows[32 + k2 // 4, pl.ds((k2 % 4) * 16, 16)] = zf16
        return 0
      lax.fori_loop(0, 32 * 4, _zt, 0, unroll=8)

    @pl.when(start >= N)
    def _():
      def _zt2(k2, _3):
        xrows[k2 // 4, pl.ds((k2 % 4) * 16, 16)] = zf16
        return 0
      lax.fori_loop(0, 64 * 4, _zt2, 0, unroll=8)

    def _cp(k, _2):
      j = k // 4
      q = (k % 4) * 16
      v = xrows[j, pl.ds(q, 16)]
      wl[j, pl.ds(q, 16)] = v
      wr[j, pl.ds(D + q, 16)] = v
      return 0
    lax.fori_loop(0, 64 * 4, _cp, 0, unroll=8)

    def _bi(j, _2):
      r = ibuf[i, pl.ds(u * 64 + j * 16, 16)] - base_row
      ok = (r >= 0) & (r < HALFW)
      sidx2[pl.ds(j * 16, 16)] = jnp.where(
          ok & ((r & 1) == 0), r >> 1, ACC_T)
      return 0
    lax.fori_loop(0, 4, _bi, 0)
    pltpu.sync_copy(wl.at[pl.ds(0, 64)], acc.at[sidx2], add=True)

    def _bo(j, _2):
      r = ibuf[i, pl.ds(u * 64 + j * 16, 16)] - base_row
      ok = (r >= 0) & (r < HALFW)
      sidx2[pl.ds(j * 16, 16)] = jnp.where(
          ok & ((r & 1) == 1), r >> 1, ACC_T)
      return 0
    lax.fori_loop(0, 4, _bo, 0)
    pltpu.sync_copy(wr.at[pl.ds(0, 64)], acc.at[sidx2], add=True)
    return 0
  lax.fori_loop(0, 2 * CPT, _sx, 0)

  def _zw2(j, _):
    wr[j // 8, pl.ds((j % 8) * 16, 16)] = zf16
    return 0
  lax.fori_loop(0, CHUNK * 8, _zw2, 0, unroll=8)

  def _sm(i, _):
    ch = s * CPT + i
    pltpu.sync_copy(mf.at[pl.ds(ch * CHUNK, CHUNK)], mvals)

    def _bi(j, _2):
      lr = ibuf[CPT + i, pl.ds(j * 16, 16)]
      sidx[pl.ds(j * 16, 16)] = jnp.where(
          lr < 0, MTRASH, 80 * (lr & 1) + (lr >> 8))
      return 0
    lax.fori_loop(0, 8, _bi, 0)

    def _st(j, _2):
      lr = ibuf[CPT + i, pl.ds(j * 16, 16)]
      col = (jnp.maximum(lr, 0) >> 1) & 127
      plsc.store_scatter(wr, [iot + j * 16, col], mvals[pl.ds(j * 16, 16)])
      return 0
    lax.fori_loop(0, CHUNK // 16, _st, 0, unroll=8)
    pltpu.sync_copy(wr, macc.at[sidx], add=True)

    def _un(j, _2):
      lr = ibuf[CPT + i, pl.ds(j * 16, 16)]
      col = (jnp.maximum(lr, 0) >> 1) & 127
      plsc.store_scatter(wr, [iot + j * 16, col], zf16)
      return 0
    lax.fori_loop(0, CHUNK // 16, _un, 0, unroll=8)
    return 0
  lax.fori_loop(0, CPT, _sm, 0)
  plsc.subcore_barrier()

  def _out(k, _):
    def _bi(j, _2):
      sidx[pl.ds(j * 16, 16)] = s * 640 + k * CHUNK + iot + j * 16
      return 0
    lax.fori_loop(0, 8, _bi, 0)
    pltpu.sync_copy(acc.at[sidx], wl)
    pltpu.sync_copy(
        wl, xexp.at[pl.ds(c * PK + s * 640 + k * CHUNK, CHUNK)])
    return 0
  lax.fori_loop(0, PK // (NTILES * CHUNK), _out, 0)

  @pl.when(s == 0)
  def _():
    pltpu.sync_copy(macc.at[pl.ds(0, CHUNK)], wr)
    pltpu.sync_copy(wr, msc.at[pl.ds(c * 160, CHUNK)])
    pltpu.sync_copy(macc.at[pl.ds(32, CHUNK)], wr)
    pltpu.sync_copy(wr, msc.at[pl.ds(c * 160 + 32, CHUNK)])


@functools.cache
def _scatter_kernel():
  mesh = plsc.VectorSubcoreMesh(core_axis_name="c", subcore_axis_name="s")
  return pl.kernel(
      _scatter_body,
      out_type=[
          jax.ShapeDtypeStruct((2 * PK, CHUNK), jnp.float32),
          jax.ShapeDtypeStruct((320, CHUNK), jnp.float32),
      ],
      mesh=mesh,
      compiler_params=pltpu.CompilerParams(needs_layout_passes=False),
      scratch_types=[
          pltpu.VMEM((2 * CPT, CHUNK), jnp.int32),
          pltpu.VMEM((64, D), jnp.float32),
          pltpu.VMEM((CHUNK, CHUNK), jnp.float32),
          pltpu.VMEM((CHUNK, CHUNK), jnp.float32),
          pltpu.VMEM((CHUNK,), jnp.float32),
          pltpu.VMEM((CHUNK,), jnp.int32),
          pltpu.VMEM((64,), jnp.int32),
          pltpu.VMEM_SHARED((PK + 8, CHUNK), jnp.float32),
          pltpu.VMEM_SHARED((MROWS, CHUNK), jnp.float32),
      ],
      name="sc_union_scatter",
  )


def _fin_body(xe_ref, ms_ref, xp_ref, tg_ref):
  x = xe_ref[...]
  xe = x[:, 0:D]
  xo = x[:, D:2 * D]
  se = ms_ref[:, 0:1]
  so = ms_ref[:, 1:2]
  te = jnp.where((se > THR) & (jnp.max(xe, axis=1, keepdims=True) > 0.0),
                 1.0, 0.0)
  to = jnp.where((so > THR) & (jnp.max(xo, axis=1, keepdims=True) > 0.0),
                 1.0, 0.0)
  xp_ref[:, 0:D] = xe * te
  xp_ref[:, D:2 * D] = xo * to
  tg_ref[:, 0:1] = te
  tg_ref[:, 1:2] = to


def _finalize(xexp_p, msc_r):
  blk = 1280
  return pl.pallas_call(
      _fin_body,
      grid=(2 * PK // blk,),
      in_specs=[
          pl.BlockSpec((blk, CHUNK), lambda i: (i, 0)),
          pl.BlockSpec((blk, 2), lambda i: (i, 0)),
      ],
      out_specs=[
          pl.BlockSpec((blk, CHUNK), lambda i: (i, 0)),
          pl.BlockSpec((blk, 2), lambda i: (i, 0)),
      ],
      out_shape=[
          jax.ShapeDtypeStruct((2 * PK, CHUNK), jnp.float32),
          jax.ShapeDtypeStruct((2 * PK, 2), jnp.float32),
      ],
      compiler_params=pltpu.CompilerParams(
          dimension_semantics=("arbitrary",)),
  )(xexp_p, msc_r)


def kernel(x_feats, x_coords, mask_feats, mask_coords):
  pad = NPAD - N
  xf = x_feats
  xc = jnp.pad(x_coords, (0, pad), constant_values=-1)
  mf = jnp.pad(mask_feats.reshape(-1), (0, pad))
  mc = jnp.pad(mask_coords, (0, pad), constant_values=-1)
  invx, invm = _rank_kernel()(xc, mc)
  xexp_p, msc = _scatter_kernel()(xf, mf, invx, invm)
  msc_r = msc.reshape(2, 2, PK).transpose(0, 2, 1).reshape(2 * PK, 2)
  xp_pk, tg2 = _finalize(xexp_p, msc_r)
  xp = xp_pk.reshape(4 * PK, D)[:U]
  tg = tg2.reshape(4 * PK)[:U].astype(jnp.bool_)
  return xp, tg
